# baseline (speedup 1.0000x reference)
import jax
import jax.numpy as jnp
from jax import lax
from jax.experimental import pallas as pl
from jax.experimental.pallas import tpu as pltpu
from jax.experimental.pallas import tpu_sc as plsc

_M = 64
_C = 200
_SUB = 100
_K = 800
_NSUB = _K // _SUB


def _positional_embedding(num_positions):
    pos = jnp.arange(num_positions)
    denom = 10000 ** jnp.linspace(0, 1, _M)
    arg = jnp.expand_dims(pos, 1) / jnp.expand_dims(denom, 0)
    pos_emb = jnp.zeros((num_positions, _M), jnp.float32)
    pos_emb = pos_emb.at[:, ::2].set(jnp.sin(arg[:, ::2]))
    pos_emb = pos_emb.at[:, 1::2].set(jnp.cos(arg[:, 1::2]))
    return pos_emb


def kernel(input, emb):
    B, C = input.shape
    assert C == _C and emb.shape[1] == _M
    N = B * C

    info = plsc.get_sparse_core_info()
    NC, NS = info.num_cores, info.num_subcores
    NW = NC * NS
    W = N // NW
    assert W % _K == 0
    G = W // _K
    assert W % _C == 0 and _K % _C == 0

    idx2d = input.reshape(N // _SUB, _SUB)
    pos = _positional_embedding(C)

    mesh = plsc.VectorSubcoreMesh(core_axis_name="c", subcore_axis_name="s")

    @pl.kernel(
        out_type=jax.ShapeDtypeStruct((N, _M), jnp.float32),
        mesh=mesh,
        compiler_params=pltpu.CompilerParams(use_tc_tiling_on_sc=False),
        scratch_types=[
            pltpu.VMEM((2, _NSUB, _SUB), jnp.int32),
            pltpu.VMEM((2, _K, _M), jnp.float32),
            pltpu.VMEM((_C, _M), jnp.float32),
            pltpu.SemaphoreType.DMA,
            pltpu.SemaphoreType.DMA,
            pltpu.SemaphoreType.DMA,
            pltpu.SemaphoreType.DMA,
        ],
    )
    def run(idx_hbm, emb_hbm, pos_hbm, out_hbm, idx_v, rows_v, pos_v,
            gsem0, gsem1, osem0, osem1):
        gsem = (gsem0, gsem1)
        osem = (osem0, osem1)
        w = lax.axis_index("s") * NC + lax.axis_index("c")
        row0 = w * (W // _SUB)
        base = w * W

        def start(g, b):
            pltpu.sync_copy(idx_hbm.at[pl.ds(row0 + g * _NSUB, _NSUB)],
                            idx_v.at[b])
            for j in range(_NSUB):
                pltpu.async_copy(emb_hbm.at[idx_v.at[b, j]],
                                 rows_v.at[b, pl.ds(j * _SUB, _SUB)],
                                 gsem[b])

        def wait_gather(b):
            pltpu.make_async_copy(emb_hbm.at[pl.ds(0, _K)], rows_v.at[b],
                                  gsem[b]).wait()

        def wait_out(b):
            pltpu.make_async_copy(rows_v.at[b], out_hbm.at[pl.ds(0, _K)],
                                  osem[b]).wait()

        def add_pos(b):
            @pl.loop(0, _C)
            def _(c):
                for j in range(_M // 16):
                    pv = pos_v[c, pl.ds(j * 16, 16)]
                    for r in range(_K // _C):
                        plsc.addupdate(
                            rows_v.at[b, r * _C + c, pl.ds(j * 16, 16)], pv)

        pltpu.sync_copy(pos_hbm, pos_v)
        start(0, 0)
        start(1, 1)

        @pl.loop(0, G, step=2)
        def _(t):
            for b in range(2):
                g = t + b
                wait_gather(b)
                add_pos(b)
                pltpu.async_copy(rows_v.at[b],
                                 out_hbm.at[pl.ds(base + g * _K, _K)],
                                 osem[b])

                @pl.when(g + 2 < G)
                def _():
                    wait_out(b)
                    start(g + 2, b)

        wait_out(0)
        wait_out(1)

    out = run(idx2d, emb, pos)
    return out.reshape(B, C, _M)


# speedup vs baseline: 1.0000x; 1.0000x over previous
import jax
import jax.numpy as jnp
from jax import lax
from jax.experimental import pallas as pl
from jax.experimental.pallas import tpu as pltpu
from jax.experimental.pallas import tpu_sc as plsc

_M = 64
_C = 200
_BW = 128
_MC = 4
_G = _C // _MC


def _positional_embedding(num_positions):
    pos = jnp.arange(num_positions)
    denom = 10000 ** jnp.linspace(0, 1, _M)
    arg = jnp.expand_dims(pos, 1) / jnp.expand_dims(denom, 0)
    pos_emb = jnp.zeros((num_positions, _M), jnp.float32)
    pos_emb = pos_emb.at[:, ::2].set(jnp.sin(arg[:, ::2]))
    pos_emb = pos_emb.at[:, 1::2].set(jnp.cos(arg[:, 1::2]))
    return pos_emb


def kernel(input, emb):
    B, C = input.shape
    assert C == _C and emb.shape[1] == _M

    info = plsc.get_sparse_core_info()
    NC, NS = info.num_cores, info.num_subcores
    NW = NC * NS
    assert B % NW == 0 and B // NW == _BW

    idx_t = input.T
    pos = _positional_embedding(C)

    mesh = plsc.VectorSubcoreMesh(core_axis_name="c", subcore_axis_name="s")

    @pl.kernel(
        out_type=jax.ShapeDtypeStruct((_C, B, _M), jnp.float32),
        mesh=mesh,
        compiler_params=pltpu.CompilerParams(use_tc_tiling_on_sc=False),
        scratch_types=[
            pltpu.VMEM((2, _MC, _BW), jnp.int32),
            pltpu.VMEM((2, _MC, _BW, _M), jnp.float32),
            pltpu.VMEM((_C, _M), jnp.float32),
            pltpu.SemaphoreType.DMA,
            pltpu.SemaphoreType.DMA,
            pltpu.SemaphoreType.DMA,
            pltpu.SemaphoreType.DMA,
        ],
    )
    def run(idx_hbm, emb_hbm, pos_hbm, out_hbm, idx_v, rows_v, pos_v,
            gsem0, gsem1, osem0, osem1):
        gsem = (gsem0, gsem1)
        osem = (osem0, osem1)
        w = lax.axis_index("s") * NC + lax.axis_index("c")
        bw = w * _BW

        def start(g, b):
            c0 = g * _MC
            pltpu.sync_copy(idx_hbm.at[pl.ds(c0, _MC), pl.ds(bw, _BW)],
                            idx_v.at[b])
            for j in range(_MC):
                pltpu.async_copy(emb_hbm.at[idx_v.at[b, j]],
                                 rows_v.at[b, j], gsem[b])

        def wait_gather(b):
            for j in range(_MC):
                pltpu.make_async_copy(emb_hbm.at[pl.ds(0, _BW)],
                                      rows_v.at[b, j], gsem[b]).wait()

        def wait_out(b):
            pltpu.make_async_copy(
                rows_v.at[b],
                out_hbm.at[pl.ds(0, _MC), pl.ds(0, _BW)], osem[b]).wait()

        def add_pos(g, b):
            c0 = g * _MC
            for j in range(_MC):
                pv = [pos_v[c0 + j, pl.ds(16 * q, 16)] for q in range(_M // 16)]

                @pl.loop(0, _BW)
                def _(r):
                    for q in range(_M // 16):
                        plsc.addupdate(
                            rows_v.at[b, j, r, pl.ds(16 * q, 16)], pv[q])

        pltpu.sync_copy(pos_hbm, pos_v)
        start(0, 0)
        start(1, 1)

        @pl.loop(0, _G, step=2)
        def _(t):
            for b in range(2):
                g = t + b
                wait_gather(b)
                add_pos(g, b)
                pltpu.async_copy(
                    rows_v.at[b],
                    out_hbm.at[pl.ds(g * _MC, _MC), pl.ds(bw, _BW)],
                    osem[b])

                @pl.when(g + 2 < _G)
                def _():
                    wait_out(b)
                    start(g + 2, b)

        wait_out(0)
        wait_out(1)

    out = run(idx_t, emb, pos)
    return out.transpose(1, 0, 2)
